# baseline (speedup 1.0000x reference)
import functools

import jax
import jax.numpy as jnp
from jax import lax
from jax.experimental import pallas as pl
from jax.experimental.pallas import tpu as pltpu
from jax.experimental.pallas import tpu_sc as plsc

NUM_NODES = 1000000
NUM_RELS = 100
D = 32
T = 16384

NC = 2
NS = 16
NW = NC * NS
PER_W = T // NW
N_STREAM = 4
STREAM_B = PER_W // N_STREAM
CHUNKS = PER_W // 16


def _sc_score(node_hbm, rel_hbm, hi_hbm, ri_hbm, ti_hbm, out_hbm,
              hidx_v, tidx_v, ridx_v, hrows_v, trows_v, rel_v, out_v,
              sem_h, sem_t):
    wid = lax.axis_index("s") * NC + lax.axis_index("c")

    pltpu.sync_copy(hi_hbm.at[wid], hidx_v)
    pltpu.sync_copy(ti_hbm.at[wid], tidx_v)
    pltpu.sync_copy(ri_hbm.at[wid], ridx_v)
    pltpu.sync_copy(rel_hbm, rel_v)

    copies = []
    for j in range(N_STREAM):
        dst = pl.ds(j * STREAM_B, STREAM_B)
        copies.append(pltpu.async_copy(
            node_hbm.at[hidx_v.at[j]], hrows_v.at[dst], sem_h))
        copies.append(pltpu.async_copy(
            node_hbm.at[tidx_v.at[j]], trows_v.at[dst], sem_t))
    for cp in copies:
        cp.wait()

    iota16 = lax.iota(jnp.int32, 16)

    def chunk_body(c, carry):
        base = c * 16
        rows = base + iota16
        ridx16 = ridx_v[pl.ds(base, 16)]
        acc = jnp.zeros((16,), jnp.float32)
        for d in range(D):
            dcol = jnp.full((16,), d, jnp.int32)
            hv = plsc.load_gather(hrows_v, [rows, dcol])
            tv = plsc.load_gather(trows_v, [rows, dcol])
            rv = plsc.load_gather(rel_v, [ridx16, dcol])
            acc = acc + hv * rv * tv
        out_v[pl.ds(base, 16)] = acc
        return carry

    lax.fori_loop(0, CHUNKS, chunk_body, 0)

    pltpu.sync_copy(out_v, out_hbm.at[wid])


def kernel(node_embedding, relational_embedding, triplets):
    tri = triplets.astype(jnp.int32)
    hi = tri[:, 0].reshape(NW, N_STREAM, STREAM_B)
    ri = tri[:, 1].reshape(NW, PER_W)
    ti = tri[:, 2].reshape(NW, N_STREAM, STREAM_B)

    mesh = plsc.VectorSubcoreMesh(core_axis_name="c", subcore_axis_name="s")
    scores = pl.kernel(
        _sc_score,
        out_type=jax.ShapeDtypeStruct((NW, PER_W), jnp.float32),
        mesh=mesh,
        compiler_params=pltpu.CompilerParams(
            needs_layout_passes=False, use_tc_tiling_on_sc=False),
        scratch_types=[
            pltpu.VMEM((N_STREAM, STREAM_B), jnp.int32),
            pltpu.VMEM((N_STREAM, STREAM_B), jnp.int32),
            pltpu.VMEM((PER_W,), jnp.int32),
            pltpu.VMEM((PER_W, D), jnp.float32),
            pltpu.VMEM((PER_W, D), jnp.float32),
            pltpu.VMEM((NUM_RELS, D), jnp.float32),
            pltpu.VMEM((PER_W,), jnp.float32),
            pltpu.SemaphoreType.DMA,
            pltpu.SemaphoreType.DMA,
        ],
    )(node_embedding, relational_embedding, hi, ri, ti)
    return scores.reshape(T)
